# baseline (speedup 1.0000x reference)
import functools

import jax
import jax.numpy as jnp
from jax import lax
from jax.experimental import pallas as pl
from jax.experimental.pallas import tpu as pltpu
from jax.experimental.pallas import tpu_sc as plsc

_L = 16


def _mlp_tc(x_pad, w1t, b1r, w2t, b2r):
    npad, in_c = x_pad.shape
    hid = w1t.shape[1]
    out_c = w2t.shape[1]
    bm = 640

    def body(x_ref, w1_ref, b1_ref, w2_ref, b2_ref, o_ref):
        h = jnp.dot(x_ref[...], w1_ref[...], preferred_element_type=jnp.float32)
        h = jnp.maximum(h + b1_ref[...], 0.0)
        o = jnp.dot(h, w2_ref[...], preferred_element_type=jnp.float32)
        o_ref[...] = o + b2_ref[...]

    return pl.pallas_call(
        body,
        grid=(npad // bm,),
        in_specs=[
            pl.BlockSpec((bm, in_c), lambda i: (i, 0)),
            pl.BlockSpec((in_c, hid), lambda i: (0, 0)),
            pl.BlockSpec((1, hid), lambda i: (0, 0)),
            pl.BlockSpec((hid, out_c), lambda i: (0, 0)),
            pl.BlockSpec((1, out_c), lambda i: (0, 0)),
        ],
        out_specs=pl.BlockSpec((bm, out_c), lambda i: (i, 0)),
        out_shape=jax.ShapeDtypeStruct((npad, out_c), jnp.float32),
    )(x_pad, w1t, b1r, w2t, b2r)


def _init_tc(degp, h, t0):
    npad, out_c = h.shape
    bm = 640

    def body(t0_ref, dp_ref, h_ref, d_ref, z_ref, hid_ref):
        deg = dp_ref[0] + dp_ref[1] + 1.0
        dinv = lax.rsqrt(deg)
        d_ref[...] = dinv
        z0 = h_ref[...] * dinv[:, 0:1]
        z_ref[...] = z0
        hid_ref[...] = t0_ref[0, 0] * z0

    return pl.pallas_call(
        body,
        grid=(npad // bm,),
        in_specs=[
            pl.BlockSpec(memory_space=pltpu.SMEM),
            pl.BlockSpec((2, bm, _L), lambda i: (0, i, 0)),
            pl.BlockSpec((bm, out_c), lambda i: (i, 0)),
        ],
        out_specs=[
            pl.BlockSpec((bm, _L), lambda i: (i, 0)),
            pl.BlockSpec((bm, out_c), lambda i: (i, 0)),
            pl.BlockSpec((bm, out_c), lambda i: (i, 0)),
        ],
        out_shape=[
            jax.ShapeDtypeStruct((npad, _L), jnp.float32),
            jax.ShapeDtypeStruct((npad, out_c), jnp.float32),
            jax.ShapeDtypeStruct((npad, out_c), jnp.float32),
        ],
    )(t0, degp, h)


def _merge_tc(part, zprev, d16, hidin, tks):
    npad, out_c = zprev.shape
    bm = 640

    def body(tk_ref, p_ref, z_ref, d_ref, h_ref, zo_ref, ho_ref):
        dinv = d_ref[:, 0:1]
        zk = (p_ref[0] + p_ref[1] + z_ref[...]) * (dinv * dinv)
        zo_ref[...] = zk
        ho_ref[...] = h_ref[...] + tk_ref[0, 0] * zk

    return pl.pallas_call(
        body,
        grid=(npad // bm,),
        in_specs=[
            pl.BlockSpec(memory_space=pltpu.SMEM),
            pl.BlockSpec((2, bm, out_c), lambda i: (0, i, 0)),
            pl.BlockSpec((bm, out_c), lambda i: (i, 0)),
            pl.BlockSpec((bm, _L), lambda i: (i, 0)),
            pl.BlockSpec((bm, out_c), lambda i: (i, 0)),
        ],
        out_specs=[
            pl.BlockSpec((bm, out_c), lambda i: (i, 0)),
            pl.BlockSpec((bm, out_c), lambda i: (i, 0)),
        ],
        out_shape=[
            jax.ShapeDtypeStruct((npad, out_c), jnp.float32),
            jax.ShapeDtypeStruct((npad, out_c), jnp.float32),
        ],
    )(tks, part, zprev, d16, hidin)


def _final_tc(part, zprev, d16, hidin, tks):
    npad, out_c = zprev.shape
    bm = 640

    def body(tk_ref, p_ref, z_ref, d_ref, h_ref, o_ref):
        dinv = d_ref[:, 0:1]
        zk = (p_ref[0] + p_ref[1] + z_ref[...]) * (dinv * dinv)
        hid = (h_ref[...] + tk_ref[0, 0] * zk) / dinv
        m = jnp.max(hid, axis=1, keepdims=True)
        e = jnp.exp(hid - m)
        s = jnp.sum(e, axis=1, keepdims=True)
        o_ref[...] = hid - m - jnp.log(s)

    return pl.pallas_call(
        body,
        grid=(npad // bm,),
        in_specs=[
            pl.BlockSpec(memory_space=pltpu.SMEM),
            pl.BlockSpec((2, bm, out_c), lambda i: (0, i, 0)),
            pl.BlockSpec((bm, out_c), lambda i: (i, 0)),
            pl.BlockSpec((bm, _L), lambda i: (i, 0)),
            pl.BlockSpec((bm, out_c), lambda i: (i, 0)),
        ],
        out_specs=pl.BlockSpec((bm, out_c), lambda i: (i, 0)),
        out_shape=jax.ShapeDtypeStruct((npad, out_c), jnp.float32),
    )(tks, part, zprev, d16, hidin)


def _edge_phase(src_hbm, dst_hbm, c, tid, zc, zn, src_i, dst_i, msg,
                semg, semi, sems, nsb, sb):
    pltpu.async_copy(src_hbm.at[c, tid, pl.ds(0, sb)], src_i.at[0], semi)
    pltpu.async_copy(dst_hbm.at[c, tid, pl.ds(0, sb)], dst_i.at[0], semi)

    @pl.loop(0, nsb)
    def _(s):
        pb = lax.rem(s, 2)
        pltpu.make_async_copy(
            src_hbm.at[c, tid, pl.ds(0, sb)], src_i.at[pb], semi).wait()
        pltpu.make_async_copy(
            dst_hbm.at[c, tid, pl.ds(0, sb)], dst_i.at[pb], semi).wait()

        @pl.when(s + 1 < nsb)
        def _():
            nb = lax.rem(s + 1, 2)
            off = (s + 1) * sb
            pltpu.async_copy(
                src_hbm.at[c, tid, pl.ds(off, sb)], src_i.at[nb], semi)
            pltpu.async_copy(
                dst_hbm.at[c, tid, pl.ds(off, sb)], dst_i.at[nb], semi)

        @pl.when(s > 0)
        def _():
            pltpu.make_async_copy(
                msg.at[1], zn.at[dst_i.at[pb, 0]], sems[1]).wait()

        descs = [None, None]
        sdescs = [None, None]
        descs[0] = pltpu.async_copy(zc.at[src_i.at[pb, 0]], msg.at[0], semg)
        for j in range(sb):
            descs[j % 2].wait()
            sdescs[j % 2] = pltpu.async_copy(
                msg.at[j % 2], zn.at[dst_i.at[pb, j]], sems[j % 2], add=True)
            if j + 1 < sb:
                if j >= 1:
                    sdescs[(j - 1) % 2].wait()
                descs[(j + 1) % 2] = pltpu.async_copy(
                    zc.at[src_i.at[pb, j + 1]], msg.at[(j + 1) % 2], semg)

    pltpu.make_async_copy(msg.at[1], zn.at[dst_i.at[0, 0]], sems[1]).wait()


def _make_deg_sc(npad, nchunk, ch):
    tiles = 16
    rpt = npad // tiles
    rc = 64
    nrc = rpt // rc
    sb = 8
    nsb = nchunk // sb
    mesh = plsc.VectorSubcoreMesh(core_axis_name="c", subcore_axis_name="s")

    @functools.partial(
        pl.kernel,
        out_type=jax.ShapeDtypeStruct((2, npad, _L), jnp.float32),
        mesh=mesh,
        compiler_params=pltpu.CompilerParams(use_tc_tiling_on_sc=False),
        scratch_types=[
            pltpu.VMEM_SHARED((npad, _L), jnp.float32),
            pltpu.VMEM((2, sb, ch), jnp.int32),
            pltpu.VMEM((ch, _L), jnp.float32),
            pltpu.VMEM((rc, _L), jnp.float32),
            pltpu.SemaphoreType.DMA,
        ],
    )
    def deg(dst_hbm, degp_hbm, dbuf, dst_i, ones_v, d_c, semd):
        c = lax.axis_index("c")
        tid = lax.axis_index("s")
        rbase = tid * rpt
        zero16 = jnp.zeros((_L,), jnp.float32)
        one16 = jnp.ones((_L,), jnp.float32)

        @pl.loop(0, rc)
        def _(r):
            d_c[r, :] = zero16

        @pl.loop(0, ch)
        def _(r):
            ones_v[r, :] = one16

        @pl.loop(0, nrc)
        def _(i):
            pltpu.sync_copy(d_c, dbuf.at[pl.ds(rbase + i * rc, rc)])

        plsc.subcore_barrier()

        pltpu.sync_copy(dst_hbm.at[c, tid, pl.ds(0, sb)], dst_i.at[0])

        @pl.loop(0, nsb)
        def _(s):
            pb = lax.rem(s, 2)

            @pl.when(s + 1 < nsb)
            def _():
                pltpu.async_copy(
                    dst_hbm.at[c, tid, pl.ds((s + 1) * sb, sb)],
                    dst_i.at[lax.rem(s + 1, 2)], semd)

            for j in range(sb):
                pltpu.sync_copy(ones_v, dbuf.at[dst_i.at[pb, j]], add=True)

            @pl.when(s + 1 < nsb)
            def _():
                pltpu.make_async_copy(
                    dst_hbm.at[c, tid, pl.ds(0, sb)],
                    dst_i.at[lax.rem(s + 1, 2)], semd).wait()

        plsc.subcore_barrier()
        pltpu.sync_copy(dbuf.at[pl.ds(rbase, rpt)],
                        degp_hbm.at[c, pl.ds(rbase, rpt)])

    return deg


def _make_edge0_sc(npad, out_c, nchunk, ch):
    tiles = 16
    rpt = npad // tiles
    rc = 64
    nrc = rpt // rc
    sb = 8
    nsb = nchunk // sb
    mesh = plsc.VectorSubcoreMesh(core_axis_name="c", subcore_axis_name="s")

    @functools.partial(
        pl.kernel,
        out_type=jax.ShapeDtypeStruct((2, npad, out_c), jnp.float32),
        mesh=mesh,
        compiler_params=pltpu.CompilerParams(use_tc_tiling_on_sc=False),
        scratch_types=[
            pltpu.VMEM_SHARED((npad, out_c), jnp.float32),
            pltpu.VMEM_SHARED((npad, out_c), jnp.float32),
            pltpu.VMEM((2, sb, ch), jnp.int32),
            pltpu.VMEM((2, sb, ch), jnp.int32),
            pltpu.VMEM((2, ch, out_c), jnp.float32),
            pltpu.VMEM((rc, out_c), jnp.float32),
            pltpu.SemaphoreType.DMA,
            pltpu.SemaphoreType.DMA,
            pltpu.SemaphoreType.DMA,
            pltpu.SemaphoreType.DMA,
        ],
    )
    def edge0(z0_hbm, src_hbm, dst_hbm, part_hbm,
              zc, zn, src_i, dst_i, msg, a_c, semg, semi, sems0, sems1):
        c = lax.axis_index("c")
        tid = lax.axis_index("s")
        rbase = tid * rpt
        zero16 = jnp.zeros((_L,), jnp.float32)
        nj = out_c // _L

        st = pltpu.async_copy(
            z0_hbm.at[pl.ds(rbase, rpt)], zc.at[pl.ds(rbase, rpt)], semg)

        @pl.loop(0, rc)
        def _(r):
            for j in range(nj):
                a_c[r, pl.ds(_L * j, _L)] = zero16

        @pl.loop(0, nrc)
        def _(i):
            pltpu.sync_copy(a_c, zn.at[pl.ds(rbase + i * rc, rc)])

        st.wait()
        plsc.subcore_barrier()
        _edge_phase(src_hbm, dst_hbm, c, tid, zc, zn, src_i, dst_i, msg,
                    semg, semi, [sems0, sems1], nsb, sb)
        plsc.subcore_barrier()
        pltpu.sync_copy(zn.at[pl.ds(rbase, rpt)],
                        part_hbm.at[c, pl.ds(rbase, rpt)])

    return edge0


def kernel(x, edge_index, W1, b1, W2, b2, temp):
    n, in_c = x.shape
    out_c = W2.shape[0]
    e = edge_index.shape[1]
    kk = temp.shape[0] - 1

    tiles = 16
    ncores = 2
    nw = tiles * ncores
    ch = 128
    npad = -(-(n + 1) // 640) * 640
    et = e // nw
    assert et * nw == e
    nchunk = -(-et // (ch * 8)) * 8
    slots = nchunk * ch
    padw = slots - et

    x_pad = jnp.concatenate(
        [x, jnp.zeros((npad - n, in_c), jnp.float32)], axis=0)
    w1t = W1.T
    w2t = W2.T
    b1r = b1.reshape(1, -1)
    b2r = b2.reshape(1, -1)

    src = edge_index[0].astype(jnp.int32).reshape(ncores, tiles, et)
    dst = edge_index[1].astype(jnp.int32).reshape(ncores, tiles, et)
    srcb = jnp.concatenate(
        [src, jnp.zeros((ncores, tiles, padw), jnp.int32)], axis=2
    ).reshape(ncores, tiles, nchunk, ch)
    dstb = jnp.concatenate(
        [dst, jnp.full((ncores, tiles, padw), n, jnp.int32)], axis=2
    ).reshape(ncores, tiles, nchunk, ch)
    tempf = temp.astype(jnp.float32)

    h = _mlp_tc(x_pad, w1t, b1r, w2t, b2r)
    degp = _make_deg_sc(npad, nchunk, ch)(dstb)
    d16, z0, hid = _init_tc(degp, h, tempf[0].reshape(1, 1))

    edge0 = _make_edge0_sc(npad, out_c, nchunk, ch)

    part = edge0(z0, srcb, dstb)
    zprev = z0
    for k in range(1, kk):
        zk, hid = _merge_tc(part, zprev, d16, hid, tempf[k].reshape(1, 1))
        part = edge0(zk, srcb, dstb)
        zprev = zk
    out = _final_tc(part, zprev, d16, hid, tempf[kk].reshape(1, 1))
    return out[:n]


# speedup vs baseline: 1.0542x; 1.0542x over previous
import functools

import jax
import jax.numpy as jnp
from jax import lax
from jax.experimental import pallas as pl
from jax.experimental.pallas import tpu as pltpu
from jax.experimental.pallas import tpu_sc as plsc

_L = 16


def _mlp_tc(x_pad, w1t, b1r, w2t, b2r):
    npad, in_c = x_pad.shape
    hid = w1t.shape[1]
    out_c = w2t.shape[1]
    bm = 640

    def body(x_ref, w1_ref, b1_ref, w2_ref, b2_ref, o_ref):
        h = jnp.dot(x_ref[...], w1_ref[...], preferred_element_type=jnp.float32)
        h = jnp.maximum(h + b1_ref[...], 0.0)
        o = jnp.dot(h, w2_ref[...], preferred_element_type=jnp.float32)
        o_ref[...] = o + b2_ref[...]

    return pl.pallas_call(
        body,
        grid=(npad // bm,),
        in_specs=[
            pl.BlockSpec((bm, in_c), lambda i: (i, 0)),
            pl.BlockSpec((in_c, hid), lambda i: (0, 0)),
            pl.BlockSpec((1, hid), lambda i: (0, 0)),
            pl.BlockSpec((hid, out_c), lambda i: (0, 0)),
            pl.BlockSpec((1, out_c), lambda i: (0, 0)),
        ],
        out_specs=pl.BlockSpec((bm, out_c), lambda i: (i, 0)),
        out_shape=jax.ShapeDtypeStruct((npad, out_c), jnp.float32),
    )(x_pad, w1t, b1r, w2t, b2r)


def _init_tc(degp, h, t0):
    npad, out_c = h.shape
    bm = 640

    def body(t0_ref, dp_ref, h_ref, d_ref, z_ref, hid_ref):
        deg = dp_ref[0] + dp_ref[1] + 1.0
        dinv = lax.rsqrt(deg)
        d_ref[...] = dinv
        z0 = h_ref[...] * dinv[:, 0:1]
        z_ref[...] = z0
        hid_ref[...] = t0_ref[0, 0] * z0

    return pl.pallas_call(
        body,
        grid=(npad // bm,),
        in_specs=[
            pl.BlockSpec(memory_space=pltpu.SMEM),
            pl.BlockSpec((2, bm, _L), lambda i: (0, i, 0)),
            pl.BlockSpec((bm, out_c), lambda i: (i, 0)),
        ],
        out_specs=[
            pl.BlockSpec((bm, _L), lambda i: (i, 0)),
            pl.BlockSpec((bm, out_c), lambda i: (i, 0)),
            pl.BlockSpec((bm, out_c), lambda i: (i, 0)),
        ],
        out_shape=[
            jax.ShapeDtypeStruct((npad, _L), jnp.float32),
            jax.ShapeDtypeStruct((npad, out_c), jnp.float32),
            jax.ShapeDtypeStruct((npad, out_c), jnp.float32),
        ],
    )(t0, degp, h)


def _hid_tc(hidin, zk, tks):
    npad, out_c = zk.shape
    bm = 640

    def body(tk_ref, h_ref, z_ref, o_ref):
        o_ref[...] = h_ref[...] + tk_ref[0, 0] * z_ref[...]

    return pl.pallas_call(
        body,
        grid=(npad // bm,),
        in_specs=[
            pl.BlockSpec(memory_space=pltpu.SMEM),
            pl.BlockSpec((bm, out_c), lambda i: (i, 0)),
            pl.BlockSpec((bm, out_c), lambda i: (i, 0)),
        ],
        out_specs=pl.BlockSpec((bm, out_c), lambda i: (i, 0)),
        out_shape=jax.ShapeDtypeStruct((npad, out_c), jnp.float32),
    )(tks, hidin, zk)


def _final_tc(part, zprev, d16, hidin, tks):
    npad, out_c = zprev.shape
    bm = 640

    def body(tk_ref, p_ref, z_ref, d_ref, h_ref, o_ref):
        dinv = d_ref[:, 0:1]
        zk = (p_ref[0] + p_ref[1] + z_ref[...]) * (dinv * dinv)
        hid = (h_ref[...] + tk_ref[0, 0] * zk) / dinv
        m = jnp.max(hid, axis=1, keepdims=True)
        e = jnp.exp(hid - m)
        s = jnp.sum(e, axis=1, keepdims=True)
        o_ref[...] = hid - m - jnp.log(s)

    return pl.pallas_call(
        body,
        grid=(npad // bm,),
        in_specs=[
            pl.BlockSpec(memory_space=pltpu.SMEM),
            pl.BlockSpec((2, bm, out_c), lambda i: (0, i, 0)),
            pl.BlockSpec((bm, out_c), lambda i: (i, 0)),
            pl.BlockSpec((bm, _L), lambda i: (i, 0)),
            pl.BlockSpec((bm, out_c), lambda i: (i, 0)),
        ],
        out_specs=pl.BlockSpec((bm, out_c), lambda i: (i, 0)),
        out_shape=jax.ShapeDtypeStruct((npad, out_c), jnp.float32),
    )(tks, part, zprev, d16, hidin)


def _edge_phase(src_hbm, dst_hbm, c, tid, zc, zn, src_i, dst_i, msg,
                semg, semi, sems, nsb, sb):
    pltpu.async_copy(src_hbm.at[c, tid, pl.ds(0, sb)], src_i.at[0], semi)
    pltpu.async_copy(dst_hbm.at[c, tid, pl.ds(0, sb)], dst_i.at[0], semi)

    @pl.loop(0, nsb)
    def _(s):
        pb = lax.rem(s, 2)
        pltpu.make_async_copy(
            src_hbm.at[c, tid, pl.ds(0, sb)], src_i.at[pb], semi).wait()
        pltpu.make_async_copy(
            dst_hbm.at[c, tid, pl.ds(0, sb)], dst_i.at[pb], semi).wait()

        @pl.when(s + 1 < nsb)
        def _():
            nb = lax.rem(s + 1, 2)
            off = (s + 1) * sb
            pltpu.async_copy(
                src_hbm.at[c, tid, pl.ds(off, sb)], src_i.at[nb], semi)
            pltpu.async_copy(
                dst_hbm.at[c, tid, pl.ds(off, sb)], dst_i.at[nb], semi)

        @pl.when(s > 0)
        def _():
            pltpu.make_async_copy(
                msg.at[1], zn.at[dst_i.at[pb, 0]], sems[1]).wait()

        descs = [None, None]
        sdescs = [None, None]
        descs[0] = pltpu.async_copy(zc.at[src_i.at[pb, 0]], msg.at[0], semg)
        for j in range(sb):
            descs[j % 2].wait()
            sdescs[j % 2] = pltpu.async_copy(
                msg.at[j % 2], zn.at[dst_i.at[pb, j]], sems[j % 2], add=True)
            if j + 1 < sb:
                if j >= 1:
                    sdescs[(j - 1) % 2].wait()
                descs[(j + 1) % 2] = pltpu.async_copy(
                    zc.at[src_i.at[pb, j + 1]], msg.at[(j + 1) % 2], semg)

    pltpu.make_async_copy(msg.at[1], zn.at[dst_i.at[0, 0]], sems[1]).wait()


def _make_deg_sc(npad, nchunk, ch):
    tiles = 16
    rpt = npad // tiles
    rc = 64
    nrc = rpt // rc
    sb = 8
    nsb = nchunk // sb
    mesh = plsc.VectorSubcoreMesh(core_axis_name="c", subcore_axis_name="s")

    @functools.partial(
        pl.kernel,
        out_type=jax.ShapeDtypeStruct((2, npad, _L), jnp.float32),
        mesh=mesh,
        compiler_params=pltpu.CompilerParams(use_tc_tiling_on_sc=False),
        scratch_types=[
            pltpu.VMEM_SHARED((npad, _L), jnp.float32),
            pltpu.VMEM((2, sb, ch), jnp.int32),
            pltpu.VMEM((ch, _L), jnp.float32),
            pltpu.VMEM((rc, _L), jnp.float32),
            pltpu.SemaphoreType.DMA,
        ],
    )
    def deg(dst_hbm, degp_hbm, dbuf, dst_i, ones_v, d_c, semd):
        c = lax.axis_index("c")
        tid = lax.axis_index("s")
        rbase = tid * rpt
        zero16 = jnp.zeros((_L,), jnp.float32)
        one16 = jnp.ones((_L,), jnp.float32)

        @pl.loop(0, rc)
        def _(r):
            d_c[r, :] = zero16

        @pl.loop(0, ch)
        def _(r):
            ones_v[r, :] = one16

        @pl.loop(0, nrc)
        def _(i):
            pltpu.sync_copy(d_c, dbuf.at[pl.ds(rbase + i * rc, rc)])

        plsc.subcore_barrier()

        pltpu.sync_copy(dst_hbm.at[c, tid, pl.ds(0, sb)], dst_i.at[0])

        @pl.loop(0, nsb)
        def _(s):
            pb = lax.rem(s, 2)

            @pl.when(s + 1 < nsb)
            def _():
                pltpu.async_copy(
                    dst_hbm.at[c, tid, pl.ds((s + 1) * sb, sb)],
                    dst_i.at[lax.rem(s + 1, 2)], semd)

            for j in range(sb):
                pltpu.sync_copy(ones_v, dbuf.at[dst_i.at[pb, j]], add=True)

            @pl.when(s + 1 < nsb)
            def _():
                pltpu.make_async_copy(
                    dst_hbm.at[c, tid, pl.ds(0, sb)],
                    dst_i.at[lax.rem(s + 1, 2)], semd).wait()

        plsc.subcore_barrier()
        pltpu.sync_copy(dbuf.at[pl.ds(rbase, rpt)],
                        degp_hbm.at[c, pl.ds(rbase, rpt)])

    return deg


def _make_edge0_sc(npad, out_c, nchunk, ch):
    tiles = 16
    rpt = npad // tiles
    rc = 64
    nrc = rpt // rc
    sb = 8
    nsb = nchunk // sb
    mesh = plsc.VectorSubcoreMesh(core_axis_name="c", subcore_axis_name="s")

    @functools.partial(
        pl.kernel,
        out_type=jax.ShapeDtypeStruct((2, npad, out_c), jnp.float32),
        mesh=mesh,
        compiler_params=pltpu.CompilerParams(use_tc_tiling_on_sc=False),
        scratch_types=[
            pltpu.VMEM_SHARED((npad, out_c), jnp.float32),
            pltpu.VMEM_SHARED((npad, out_c), jnp.float32),
            pltpu.VMEM((2, sb, ch), jnp.int32),
            pltpu.VMEM((2, sb, ch), jnp.int32),
            pltpu.VMEM((2, ch, out_c), jnp.float32),
            pltpu.VMEM((rc, out_c), jnp.float32),
            pltpu.SemaphoreType.DMA,
            pltpu.SemaphoreType.DMA,
            pltpu.SemaphoreType.DMA,
            pltpu.SemaphoreType.DMA,
        ],
    )
    def edge0(z0_hbm, src_hbm, dst_hbm, part_hbm,
              zc, zn, src_i, dst_i, msg, a_c, semg, semi, sems0, sems1):
        c = lax.axis_index("c")
        tid = lax.axis_index("s")
        rbase = tid * rpt
        zero16 = jnp.zeros((_L,), jnp.float32)
        nj = out_c // _L

        st = pltpu.async_copy(
            z0_hbm.at[pl.ds(rbase, rpt)], zc.at[pl.ds(rbase, rpt)], semg)

        @pl.loop(0, rc)
        def _(r):
            for j in range(nj):
                a_c[r, pl.ds(_L * j, _L)] = zero16

        @pl.loop(0, nrc)
        def _(i):
            pltpu.sync_copy(a_c, zn.at[pl.ds(rbase + i * rc, rc)])

        st.wait()
        plsc.subcore_barrier()
        _edge_phase(src_hbm, dst_hbm, c, tid, zc, zn, src_i, dst_i, msg,
                    semg, semi, [sems0, sems1], nsb, sb)
        plsc.subcore_barrier()
        pltpu.sync_copy(zn.at[pl.ds(rbase, rpt)],
                        part_hbm.at[c, pl.ds(rbase, rpt)])

    return edge0


def _make_edge_sc(npad, out_c, nchunk, ch):
    tiles = 16
    rpt = npad // tiles
    rc = 64
    nrc = rpt // rc
    hrc = nrc // 2
    sb = 8
    nsb = nchunk // sb
    nj = out_c // _L
    mesh = plsc.VectorSubcoreMesh(core_axis_name="c", subcore_axis_name="s")

    @functools.partial(
        pl.kernel,
        out_type=[
            jax.ShapeDtypeStruct((2, npad, out_c), jnp.float32),
            jax.ShapeDtypeStruct((npad, out_c), jnp.float32),
        ],
        mesh=mesh,
        compiler_params=pltpu.CompilerParams(use_tc_tiling_on_sc=False),
        scratch_types=[
            pltpu.VMEM_SHARED((npad, out_c), jnp.float32),
            pltpu.VMEM_SHARED((npad, out_c), jnp.float32),
            pltpu.VMEM((2, sb, ch), jnp.int32),
            pltpu.VMEM((2, sb, ch), jnp.int32),
            pltpu.VMEM((2, ch, out_c), jnp.float32),
            pltpu.VMEM((rc, out_c), jnp.float32),
            pltpu.VMEM((rc, out_c), jnp.float32),
            pltpu.VMEM((rc, out_c), jnp.float32),
            pltpu.VMEM((rc, _L), jnp.float32),
            pltpu.SemaphoreType.DMA,
            pltpu.SemaphoreType.DMA,
            pltpu.SemaphoreType.DMA,
            pltpu.SemaphoreType.DMA,
            pltpu.SemaphoreType.DMA,
        ],
    )
    def edge(zprev_hbm, pin_hbm, d16_hbm,
             src_hbm, dst_hbm, part_hbm, zout_hbm,
             zc, zn, src_i, dst_i, msg, a_c, z_c, h_c, d_c,
             semg, semi, sems0, sems1, semst):
        c = lax.axis_index("c")
        tid = lax.axis_index("s")
        rbase = tid * rpt
        zero16 = jnp.zeros((_L,), jnp.float32)

        @pl.loop(0, nrc)
        def _(i):
            b = rbase + i * rc

            @pl.when(i > 0)
            def _():
                pltpu.make_async_copy(
                    a_c, zc.at[pl.ds(b - rc, rc)], semst).wait()

            la = pltpu.async_copy(pin_hbm.at[0, pl.ds(b, rc)], a_c, semg)
            lz = pltpu.async_copy(pin_hbm.at[1, pl.ds(b, rc)], z_c, semi)
            lh = pltpu.async_copy(zprev_hbm.at[pl.ds(b, rc)], h_c, sems0)
            ld = pltpu.async_copy(d16_hbm.at[pl.ds(b, rc)], d_c, sems1)
            la.wait()
            lz.wait()
            lh.wait()
            ld.wait()

            @pl.loop(0, rc)
            def _(r):
                y = d_c[r, :]
                d2 = y * y
                for j in range(nj):
                    sl = pl.ds(_L * j, _L)
                    a_c[r, sl] = (a_c[r, sl] + z_c[r, sl] + h_c[r, sl]) * d2

            pltpu.async_copy(a_c, zc.at[pl.ds(b, rc)], semst)

        pltpu.make_async_copy(
            a_c, zc.at[pl.ds(rbase + (nrc - 1) * rc, rc)], semst).wait()

        hbase = rbase + c * (hrc * rc)
        pltpu.async_copy(zc.at[pl.ds(hbase, hrc * rc)],
                         zout_hbm.at[pl.ds(hbase, hrc * rc)], semst)

        @pl.loop(0, rc)
        def _(r):
            for j in range(nj):
                z_c[r, pl.ds(_L * j, _L)] = zero16

        @pl.loop(0, nrc)
        def _(i):
            pltpu.sync_copy(z_c, zn.at[pl.ds(rbase + i * rc, rc)])

        plsc.subcore_barrier()
        _edge_phase(src_hbm, dst_hbm, c, tid, zc, zn, src_i, dst_i, msg,
                    semg, semi, [sems0, sems1], nsb, sb)
        plsc.subcore_barrier()
        pltpu.sync_copy(zn.at[pl.ds(rbase, rpt)],
                        part_hbm.at[c, pl.ds(rbase, rpt)])
        pltpu.make_async_copy(
            zc.at[pl.ds(hbase, hrc * rc)],
            zout_hbm.at[pl.ds(hbase, hrc * rc)], semst).wait()

    return edge


def kernel(x, edge_index, W1, b1, W2, b2, temp):
    n, in_c = x.shape
    out_c = W2.shape[0]
    e = edge_index.shape[1]
    kk = temp.shape[0] - 1

    tiles = 16
    ncores = 2
    nw = tiles * ncores
    ch = 128
    npad = -(-(n + 1) // 640) * 640
    et = e // nw
    assert et * nw == e
    nchunk = -(-et // (ch * 8)) * 8
    slots = nchunk * ch
    padw = slots - et

    x_pad = jnp.concatenate(
        [x, jnp.zeros((npad - n, in_c), jnp.float32)], axis=0)
    w1t = W1.T
    w2t = W2.T
    b1r = b1.reshape(1, -1)
    b2r = b2.reshape(1, -1)

    src = edge_index[0].astype(jnp.int32).reshape(ncores, tiles, et)
    dst = edge_index[1].astype(jnp.int32).reshape(ncores, tiles, et)
    srcb = jnp.concatenate(
        [src, jnp.zeros((ncores, tiles, padw), jnp.int32)], axis=2
    ).reshape(ncores, tiles, nchunk, ch)
    dstb = jnp.concatenate(
        [dst, jnp.full((ncores, tiles, padw), n, jnp.int32)], axis=2
    ).reshape(ncores, tiles, nchunk, ch)
    tempf = temp.astype(jnp.float32)

    h = _mlp_tc(x_pad, w1t, b1r, w2t, b2r)
    degp = _make_deg_sc(npad, nchunk, ch)(dstb)
    d16, z0, hid = _init_tc(degp, h, tempf[0].reshape(1, 1))

    edge0 = _make_edge0_sc(npad, out_c, nchunk, ch)
    edge = _make_edge_sc(npad, out_c, nchunk, ch)

    part = edge0(z0, srcb, dstb)
    zprev = z0
    for k in range(1, kk):
        part, zk = edge(zprev, part, d16, srcb, dstb)
        hid = _hid_tc(hid, zk, tempf[k].reshape(1, 1))
        zprev = zk
    out = _final_tc(part, zprev, d16, hid, tempf[kk].reshape(1, 1))
    return out[:n]


# speedup vs baseline: 1.0688x; 1.0139x over previous
import functools

import jax
import jax.numpy as jnp
from jax import lax
from jax.experimental import pallas as pl
from jax.experimental.pallas import tpu as pltpu
from jax.experimental.pallas import tpu_sc as plsc

_L = 16


def _mlp_tc(x_pad, w1t, b1r, w2t, b2r):
    npad, in_c = x_pad.shape
    hid = w1t.shape[1]
    out_c = w2t.shape[1]
    bm = 640

    def body(x_ref, w1_ref, b1_ref, w2_ref, b2_ref, o_ref):
        h = jnp.dot(x_ref[...], w1_ref[...], preferred_element_type=jnp.float32)
        h = jnp.maximum(h + b1_ref[...], 0.0)
        o = jnp.dot(h, w2_ref[...], preferred_element_type=jnp.float32)
        o_ref[...] = o + b2_ref[...]

    return pl.pallas_call(
        body,
        grid=(npad // bm,),
        in_specs=[
            pl.BlockSpec((bm, in_c), lambda i: (i, 0)),
            pl.BlockSpec((in_c, hid), lambda i: (0, 0)),
            pl.BlockSpec((1, hid), lambda i: (0, 0)),
            pl.BlockSpec((hid, out_c), lambda i: (0, 0)),
            pl.BlockSpec((1, out_c), lambda i: (0, 0)),
        ],
        out_specs=pl.BlockSpec((bm, out_c), lambda i: (i, 0)),
        out_shape=jax.ShapeDtypeStruct((npad, out_c), jnp.float32),
    )(x_pad, w1t, b1r, w2t, b2r)


def _init_tc(degp, h, t0):
    npad, out_c = h.shape
    bm = 640

    def body(t0_ref, dp_ref, h_ref, d_ref, z_ref, hid_ref):
        deg = dp_ref[0] + dp_ref[1] + 1.0
        dinv = lax.rsqrt(deg)
        d_ref[...] = dinv
        z0 = h_ref[...] * dinv[:, 0:1]
        z_ref[...] = z0
        hid_ref[...] = t0_ref[0, 0] * z0

    return pl.pallas_call(
        body,
        grid=(npad // bm,),
        in_specs=[
            pl.BlockSpec(memory_space=pltpu.SMEM),
            pl.BlockSpec((2, bm, _L), lambda i: (0, i, 0)),
            pl.BlockSpec((bm, out_c), lambda i: (i, 0)),
        ],
        out_specs=[
            pl.BlockSpec((bm, _L), lambda i: (i, 0)),
            pl.BlockSpec((bm, out_c), lambda i: (i, 0)),
            pl.BlockSpec((bm, out_c), lambda i: (i, 0)),
        ],
        out_shape=[
            jax.ShapeDtypeStruct((npad, _L), jnp.float32),
            jax.ShapeDtypeStruct((npad, out_c), jnp.float32),
            jax.ShapeDtypeStruct((npad, out_c), jnp.float32),
        ],
    )(t0, degp, h)


def _hid_tc(hidin, zk, tks):
    npad, out_c = zk.shape
    bm = 640

    def body(tk_ref, h_ref, z_ref, o_ref):
        o_ref[...] = h_ref[...] + tk_ref[0, 0] * z_ref[...]

    return pl.pallas_call(
        body,
        grid=(npad // bm,),
        in_specs=[
            pl.BlockSpec(memory_space=pltpu.SMEM),
            pl.BlockSpec((bm, out_c), lambda i: (i, 0)),
            pl.BlockSpec((bm, out_c), lambda i: (i, 0)),
        ],
        out_specs=pl.BlockSpec((bm, out_c), lambda i: (i, 0)),
        out_shape=jax.ShapeDtypeStruct((npad, out_c), jnp.float32),
    )(tks, hidin, zk)


def _final_tc(part, zprev, d16, hidin, tks):
    npad, out_c = zprev.shape
    bm = 640

    def body(tk_ref, p_ref, z_ref, d_ref, h_ref, o_ref):
        dinv = d_ref[:, 0:1]
        zk = (p_ref[0] + p_ref[1] + z_ref[...]) * (dinv * dinv)
        hid = (h_ref[...] + tk_ref[0, 0] * zk) / dinv
        m = jnp.max(hid, axis=1, keepdims=True)
        e = jnp.exp(hid - m)
        s = jnp.sum(e, axis=1, keepdims=True)
        o_ref[...] = hid - m - jnp.log(s)

    return pl.pallas_call(
        body,
        grid=(npad // bm,),
        in_specs=[
            pl.BlockSpec(memory_space=pltpu.SMEM),
            pl.BlockSpec((2, bm, out_c), lambda i: (0, i, 0)),
            pl.BlockSpec((bm, out_c), lambda i: (i, 0)),
            pl.BlockSpec((bm, _L), lambda i: (i, 0)),
            pl.BlockSpec((bm, out_c), lambda i: (i, 0)),
        ],
        out_specs=pl.BlockSpec((bm, out_c), lambda i: (i, 0)),
        out_shape=jax.ShapeDtypeStruct((npad, out_c), jnp.float32),
    )(tks, part, zprev, d16, hidin)


def _edge_phase(src_hbm, dst_hbm, c, tid, zc, zn, src_i, dst_i, msg,
                semg, semi, sems, nsb, sb):
    pltpu.async_copy(src_hbm.at[c, tid, pl.ds(0, sb)], src_i.at[0], semi)
    pltpu.async_copy(dst_hbm.at[c, tid, pl.ds(0, sb)], dst_i.at[0], semi)

    @pl.loop(0, nsb)
    def _(s):
        pb = lax.rem(s, 2)
        pltpu.make_async_copy(
            src_hbm.at[c, tid, pl.ds(0, sb)], src_i.at[pb], semi).wait()
        pltpu.make_async_copy(
            dst_hbm.at[c, tid, pl.ds(0, sb)], dst_i.at[pb], semi).wait()

        @pl.when(s + 1 < nsb)
        def _():
            nb = lax.rem(s + 1, 2)
            off = (s + 1) * sb
            pltpu.async_copy(
                src_hbm.at[c, tid, pl.ds(off, sb)], src_i.at[nb], semi)
            pltpu.async_copy(
                dst_hbm.at[c, tid, pl.ds(off, sb)], dst_i.at[nb], semi)

        @pl.when(s > 0)
        def _():
            pltpu.make_async_copy(
                msg.at[1], zn.at[dst_i.at[pb, 0]], sems[1]).wait()

        descs = [None, None]
        sdescs = [None, None]
        descs[0] = pltpu.async_copy(zc.at[src_i.at[pb, 0]], msg.at[0], semg)
        for j in range(sb):
            descs[j % 2].wait()
            sdescs[j % 2] = pltpu.async_copy(
                msg.at[j % 2], zn.at[dst_i.at[pb, j]], sems[j % 2], add=True)
            if j + 1 < sb:
                if j >= 1:
                    sdescs[(j - 1) % 2].wait()
                descs[(j + 1) % 2] = pltpu.async_copy(
                    zc.at[src_i.at[pb, j + 1]], msg.at[(j + 1) % 2], semg)

    pltpu.make_async_copy(msg.at[1], zn.at[dst_i.at[0, 0]], sems[1]).wait()


def _make_deg_sc(npad, nchunk, ch):
    tiles = 16
    rpt = npad // tiles
    rc = 64
    nrc = rpt // rc
    sb = 8
    nsb = nchunk // sb
    mesh = plsc.VectorSubcoreMesh(core_axis_name="c", subcore_axis_name="s")

    @functools.partial(
        pl.kernel,
        out_type=jax.ShapeDtypeStruct((2, npad, _L), jnp.float32),
        mesh=mesh,
        compiler_params=pltpu.CompilerParams(use_tc_tiling_on_sc=False),
        scratch_types=[
            pltpu.VMEM_SHARED((npad, _L), jnp.float32),
            pltpu.VMEM((2, sb, ch), jnp.int32),
            pltpu.VMEM((ch, _L), jnp.float32),
            pltpu.VMEM((rc, _L), jnp.float32),
            pltpu.SemaphoreType.DMA,
        ],
    )
    def deg(dst_hbm, degp_hbm, dbuf, dst_i, ones_v, d_c, semd):
        c = lax.axis_index("c")
        tid = lax.axis_index("s")
        rbase = tid * rpt
        zero16 = jnp.zeros((_L,), jnp.float32)
        one16 = jnp.ones((_L,), jnp.float32)

        @pl.loop(0, rc)
        def _(r):
            d_c[r, :] = zero16

        @pl.loop(0, ch)
        def _(r):
            ones_v[r, :] = one16

        @pl.loop(0, nrc)
        def _(i):
            pltpu.sync_copy(d_c, dbuf.at[pl.ds(rbase + i * rc, rc)])

        plsc.subcore_barrier()

        pltpu.sync_copy(dst_hbm.at[c, tid, pl.ds(0, sb)], dst_i.at[0])

        @pl.loop(0, nsb)
        def _(s):
            pb = lax.rem(s, 2)

            @pl.when(s + 1 < nsb)
            def _():
                pltpu.async_copy(
                    dst_hbm.at[c, tid, pl.ds((s + 1) * sb, sb)],
                    dst_i.at[lax.rem(s + 1, 2)], semd)

            for j in range(sb):
                pltpu.sync_copy(ones_v, dbuf.at[dst_i.at[pb, j]], add=True)

            @pl.when(s + 1 < nsb)
            def _():
                pltpu.make_async_copy(
                    dst_hbm.at[c, tid, pl.ds(0, sb)],
                    dst_i.at[lax.rem(s + 1, 2)], semd).wait()

        plsc.subcore_barrier()
        pltpu.sync_copy(dbuf.at[pl.ds(rbase, rpt)],
                        degp_hbm.at[c, pl.ds(rbase, rpt)])

    return deg


def _make_edge0_sc(npad, out_c, nchunk, ch):
    tiles = 16
    rpt = npad // tiles
    rc = 64
    nrc = rpt // rc
    sb = 8
    nsb = nchunk // sb
    mesh = plsc.VectorSubcoreMesh(core_axis_name="c", subcore_axis_name="s")

    @functools.partial(
        pl.kernel,
        out_type=jax.ShapeDtypeStruct((2, npad, out_c), jnp.float32),
        mesh=mesh,
        compiler_params=pltpu.CompilerParams(use_tc_tiling_on_sc=False),
        scratch_types=[
            pltpu.VMEM_SHARED((npad, out_c), jnp.float32),
            pltpu.VMEM_SHARED((npad, out_c), jnp.float32),
            pltpu.VMEM((2, sb, ch), jnp.int32),
            pltpu.VMEM((2, sb, ch), jnp.int32),
            pltpu.VMEM((2, ch, out_c), jnp.float32),
            pltpu.VMEM((rc, out_c), jnp.float32),
            pltpu.SemaphoreType.DMA,
            pltpu.SemaphoreType.DMA,
            pltpu.SemaphoreType.DMA,
            pltpu.SemaphoreType.DMA,
        ],
    )
    def edge0(z0_hbm, src_hbm, dst_hbm, part_hbm,
              zc, zn, src_i, dst_i, msg, a_c, semg, semi, sems0, sems1):
        c = lax.axis_index("c")
        tid = lax.axis_index("s")
        rbase = tid * rpt
        zero16 = jnp.zeros((_L,), jnp.float32)
        nj = out_c // _L

        st = pltpu.async_copy(
            z0_hbm.at[pl.ds(rbase, rpt)], zc.at[pl.ds(rbase, rpt)], semg)

        @pl.loop(0, rc)
        def _(r):
            for j in range(nj):
                a_c[r, pl.ds(_L * j, _L)] = zero16

        @pl.loop(0, nrc)
        def _(i):
            pltpu.sync_copy(a_c, zn.at[pl.ds(rbase + i * rc, rc)])

        st.wait()
        plsc.subcore_barrier()
        _edge_phase(src_hbm, dst_hbm, c, tid, zc, zn, src_i, dst_i, msg,
                    semg, semi, [sems0, sems1], nsb, sb)
        plsc.subcore_barrier()
        pltpu.sync_copy(zn.at[pl.ds(rbase, rpt)],
                        part_hbm.at[c, pl.ds(rbase, rpt)])

    return edge0


def _make_edge_sc(npad, out_c, nchunk, ch):
    tiles = 16
    rpt = npad // tiles
    rc = 64
    nrc = rpt // rc
    hrc = nrc // 2
    sb = 8
    nsb = nchunk // sb
    nj = out_c // _L
    mesh = plsc.VectorSubcoreMesh(core_axis_name="c", subcore_axis_name="s")

    @functools.partial(
        pl.kernel,
        out_type=[
            jax.ShapeDtypeStruct((2, npad, out_c), jnp.float32),
            jax.ShapeDtypeStruct((npad, out_c), jnp.float32),
        ],
        mesh=mesh,
        compiler_params=pltpu.CompilerParams(use_tc_tiling_on_sc=False),
        scratch_types=[
            pltpu.VMEM_SHARED((npad, out_c), jnp.float32),
            pltpu.VMEM_SHARED((npad, out_c), jnp.float32),
            pltpu.VMEM((2, sb, ch), jnp.int32),
            pltpu.VMEM((2, sb, ch), jnp.int32),
            pltpu.VMEM((2, ch, out_c), jnp.float32),
            pltpu.VMEM((rc, out_c), jnp.float32),
            pltpu.VMEM((rc, out_c), jnp.float32),
            pltpu.VMEM((rc, out_c), jnp.float32),
            pltpu.VMEM((rc, _L), jnp.float32),
            pltpu.VMEM((rc, out_c), jnp.float32),
            pltpu.SemaphoreType.DMA,
            pltpu.SemaphoreType.DMA,
            pltpu.SemaphoreType.DMA,
            pltpu.SemaphoreType.DMA,
            pltpu.SemaphoreType.DMA,
            pltpu.SemaphoreType.DMA,
        ],
    )
    def edge(zprev_hbm, pin_hbm, d16_hbm,
             src_hbm, dst_hbm, part_hbm, zout_hbm,
             zc, zn, src_i, dst_i, msg, a_c, z_c, h_c, d_c, zb,
             semg, semi, sems0, sems1, semst, semz):
        c = lax.axis_index("c")
        tid = lax.axis_index("s")
        rbase = tid * rpt
        zero16 = jnp.zeros((_L,), jnp.float32)

        @pl.loop(0, rc)
        def _(r):
            for j in range(nj):
                zb[r, pl.ds(_L * j, _L)] = zero16

        @pl.loop(0, nrc)
        def _(i):
            pltpu.async_copy(zb, zn.at[pl.ds(rbase + i * rc, rc)], semz)

        @pl.loop(0, nrc)
        def _(i):
            b = rbase + i * rc

            @pl.when(i > 0)
            def _():
                pltpu.make_async_copy(
                    a_c, zc.at[pl.ds(b - rc, rc)], semst).wait()

            la = pltpu.async_copy(pin_hbm.at[0, pl.ds(b, rc)], a_c, semg)
            lz = pltpu.async_copy(pin_hbm.at[1, pl.ds(b, rc)], z_c, semi)
            lh = pltpu.async_copy(zprev_hbm.at[pl.ds(b, rc)], h_c, sems0)
            ld = pltpu.async_copy(d16_hbm.at[pl.ds(b, rc)], d_c, sems1)
            la.wait()
            lz.wait()
            lh.wait()
            ld.wait()

            @pl.loop(0, rc)
            def _(r):
                y = d_c[r, :]
                d2 = y * y
                for j in range(nj):
                    sl = pl.ds(_L * j, _L)
                    a_c[r, sl] = (a_c[r, sl] + z_c[r, sl] + h_c[r, sl]) * d2

            pltpu.async_copy(a_c, zc.at[pl.ds(b, rc)], semst)

        pltpu.make_async_copy(
            a_c, zc.at[pl.ds(rbase + (nrc - 1) * rc, rc)], semst).wait()

        hbase = rbase + c * (hrc * rc)
        pltpu.async_copy(zc.at[pl.ds(hbase, hrc * rc)],
                         zout_hbm.at[pl.ds(hbase, hrc * rc)], semst)

        @pl.loop(0, nrc)
        def _(i):
            pltpu.make_async_copy(
                zb, zn.at[pl.ds(rbase + i * rc, rc)], semz).wait()

        plsc.subcore_barrier()
        _edge_phase(src_hbm, dst_hbm, c, tid, zc, zn, src_i, dst_i, msg,
                    semg, semi, [sems0, sems1], nsb, sb)
        plsc.subcore_barrier()
        pltpu.sync_copy(zn.at[pl.ds(rbase, rpt)],
                        part_hbm.at[c, pl.ds(rbase, rpt)])
        pltpu.make_async_copy(
            zc.at[pl.ds(hbase, hrc * rc)],
            zout_hbm.at[pl.ds(hbase, hrc * rc)], semst).wait()

    return edge


def kernel(x, edge_index, W1, b1, W2, b2, temp):
    n, in_c = x.shape
    out_c = W2.shape[0]
    e = edge_index.shape[1]
    kk = temp.shape[0] - 1

    tiles = 16
    ncores = 2
    nw = tiles * ncores
    ch = 128
    npad = -(-(n + 1) // 640) * 640
    et = e // nw
    assert et * nw == e
    nchunk = -(-et // (ch * 8)) * 8
    slots = nchunk * ch
    padw = slots - et

    x_pad = jnp.concatenate(
        [x, jnp.zeros((npad - n, in_c), jnp.float32)], axis=0)
    w1t = W1.T
    w2t = W2.T
    b1r = b1.reshape(1, -1)
    b2r = b2.reshape(1, -1)

    src = edge_index[0].astype(jnp.int32).reshape(ncores, tiles, et)
    dst = edge_index[1].astype(jnp.int32).reshape(ncores, tiles, et)
    srcb = jnp.concatenate(
        [src, jnp.zeros((ncores, tiles, padw), jnp.int32)], axis=2
    ).reshape(ncores, tiles, nchunk, ch)
    dstb = jnp.concatenate(
        [dst, jnp.full((ncores, tiles, padw), n, jnp.int32)], axis=2
    ).reshape(ncores, tiles, nchunk, ch)
    tempf = temp.astype(jnp.float32)

    h = _mlp_tc(x_pad, w1t, b1r, w2t, b2r)
    degp = _make_deg_sc(npad, nchunk, ch)(dstb)
    d16, z0, hid = _init_tc(degp, h, tempf[0].reshape(1, 1))

    edge0 = _make_edge0_sc(npad, out_c, nchunk, ch)
    edge = _make_edge_sc(npad, out_c, nchunk, ch)

    part = edge0(z0, srcb, dstb)
    zprev = z0
    for k in range(1, kk):
        part, zk = edge(zprev, part, d16, srcb, dstb)
        hid = _hid_tc(hid, zk, tempf[k].reshape(1, 1))
        zprev = zk
    out = _final_tc(part, zprev, d16, hid, tempf[kk].reshape(1, 1))
    return out[:n]
